# baseline (speedup 1.0000x reference)
import functools

import jax
import jax.numpy as jnp
from jax import lax
from jax.experimental import pallas as pl
from jax.experimental.pallas import tpu as pltpu
from jax.experimental.pallas import tpu_sc as plsc

_BATCH = 16384
_EMB = 64
_NODES = 1000000
_BLK = 128

_NUM_CORES = 2
_NUM_SUBCORES = 16
_NUM_WORKERS = _NUM_CORES * _NUM_SUBCORES
_LANES = 16
_NGROUP = _BATCH // _LANES

_NBLOCKS = -(-_NODES // _BLK)
_BLK_PER_W = -(-_NBLOCKS // _NUM_WORKERS)
_NODES_PER_W = _BLK_PER_W * _BLK

_mesh = plsc.VectorSubcoreMesh(
    core_axis_name="c",
    subcore_axis_name="s",
    num_cores=_NUM_CORES,
    num_subcores=_NUM_SUBCORES,
)


@functools.partial(
    pl.kernel,
    out_type=jax.ShapeDtypeStruct((_BATCH, _EMB), jnp.float32),
    mesh=_mesh,
    scratch_types=[
        pltpu.VMEM((_BATCH,), jnp.int32),
        pltpu.VMEM((_EMB, _BLK), jnp.float32),
        pltpu.VMEM((1, _EMB), jnp.float32),
        pltpu.SemaphoreType.DMA,
    ],
    compiler_params=pltpu.CompilerParams(needs_layout_passes=False),
)
def _sc_gather(table_hbm, idx_hbm, out_hbm, idx_v, blk_v, row_v, sem):
    wid = lax.axis_index("s") * _NUM_CORES + lax.axis_index("c")
    lo = wid * _NODES_PER_W
    hi = lo + _NODES_PER_W
    pltpu.sync_copy(idx_hbm, idx_v)

    def group(g, _):
        gbase = pl.multiple_of(g * _LANES, _LANES)
        i16 = idx_v[pl.ds(gbase, _LANES)]
        in_range = jnp.logical_and(i16 >= lo, i16 < hi).astype(jnp.int32)

        for j in range(_LANES):
            @pl.when(in_range[j] == 1)
            def _():
                i = i16[j]
                cbase = pl.multiple_of((i // _BLK) * _BLK, _BLK)
                ci = i - cbase
                pltpu.async_copy(
                    table_hbm.at[:, pl.ds(cbase, _BLK)], blk_v, sem
                ).wait()
                ci16 = jnp.full((_LANES,), ci, jnp.int32)
                for a in range(_EMB // _LANES):
                    e16 = lax.iota(jnp.int32, _LANES) + a * _LANES
                    row_v[0, pl.ds(a * _LANES, _LANES)] = plsc.load_gather(
                        blk_v, [e16, ci16]
                    )
                pltpu.sync_copy(row_v, out_hbm.at[pl.ds(gbase + j, 1)])

        return ()

    lax.fori_loop(0, _NGROUP, group, (), unroll=False)


@jax.jit
def kernel(idx, emb_weight):
    return _sc_gather(emb_weight.T, idx.astype(jnp.int32))


# speedup vs baseline: 4.2480x; 4.2480x over previous
import functools

import jax
import jax.numpy as jnp
from jax import lax
from jax.experimental import pallas as pl
from jax.experimental.pallas import tpu as pltpu
from jax.experimental.pallas import tpu_sc as plsc

_BATCH = 16384
_EMB = 64
_NODES = 1000000
_BLK = 128

_NUM_CORES = 2
_NUM_SUBCORES = 16
_NUM_WORKERS = _NUM_CORES * _NUM_SUBCORES
_LANES = 16
_NGROUP = _BATCH // _LANES
_RING = 4

_NBLOCKS = -(-_NODES // _BLK)
_BLK_PER_W = -(-_NBLOCKS // _NUM_WORKERS)
_NODES_PER_W = _BLK_PER_W * _BLK

_mesh = plsc.VectorSubcoreMesh(
    core_axis_name="c",
    subcore_axis_name="s",
    num_cores=_NUM_CORES,
    num_subcores=_NUM_SUBCORES,
)


@functools.partial(
    pl.kernel,
    out_type=jax.ShapeDtypeStruct((_BATCH, _EMB), jnp.float32),
    mesh=_mesh,
    scratch_types=[
        pltpu.VMEM((_BATCH,), jnp.int32),
        pltpu.VMEM((_BATCH,), jnp.int32),
        pltpu.VMEM((_BATCH,), jnp.int32),
        [pltpu.VMEM((_EMB, _BLK), jnp.float32) for _ in range(_RING)],
        pltpu.VMEM((1, _EMB), jnp.float32),
        [pltpu.SemaphoreType.DMA for _ in range(_RING)],
    ],
    compiler_params=pltpu.CompilerParams(needs_layout_passes=False),
)
def _sc_gather(table_hbm, idx_hbm, out_hbm, idx_v, klist_v, ilist_v,
               blk_bufs, row_v, sems):
    wid = lax.axis_index("s") * _NUM_CORES + lax.axis_index("c")
    lo = wid * _NODES_PER_W
    hi = lo + _NODES_PER_W
    pltpu.sync_copy(idx_hbm, idx_v)

    def scan_group(g, off):
        gbase = pl.multiple_of(g * _LANES, _LANES)
        i16 = idx_v[pl.ds(gbase, _LANES)]
        mask = jnp.logical_and(i16 >= lo, i16 < hi)
        m32 = mask.astype(jnp.int32)
        inc = plsc.cumsum(m32)
        pos16 = off + inc - m32
        k16 = lax.iota(jnp.int32, _LANES) + gbase
        plsc.store_scatter(klist_v, [pos16], k16, mask=mask)
        plsc.store_scatter(ilist_v, [pos16], i16, mask=mask)
        cnt16 = plsc.all_reduce_population_count(mask)
        return off + cnt16[0]

    cnt = lax.fori_loop(0, _NGROUP, scan_group, jnp.int32(0), unroll=False)

    def fetch(cb, slot):
        return pltpu.async_copy(
            table_hbm.at[:, pl.ds(pl.multiple_of(cb, _BLK), _BLK)],
            blk_bufs[slot], sems[slot],
        )

    def select_group(g, _):
        gbase = g * _LANES
        kv = klist_v[pl.ds(gbase, _LANES)]
        iv = ilist_v[pl.ds(gbase, _LANES)]
        cb16 = (iv // _BLK) * _BLK
        ci16 = iv - cb16

        def finish(j):
            slot = j % _RING
            pltpu.make_async_copy(
                table_hbm.at[:, pl.ds(pl.multiple_of(cb16[j], _BLK), _BLK)],
                blk_bufs[slot], sems[slot],
            ).wait()
            ci_s = jnp.full((_LANES,), ci16[j], jnp.int32)
            for a in range(_EMB // _LANES):
                e16 = lax.iota(jnp.int32, _LANES) + a * _LANES
                row_v[0, pl.ds(a * _LANES, _LANES)] = plsc.load_gather(
                    blk_bufs[slot], [e16, ci_s]
                )
            pltpu.sync_copy(row_v, out_hbm.at[pl.ds(kv[j], 1)])

        for j in range(_LANES):
            if j >= _RING:
                @pl.when(gbase + j - _RING < cnt)
                def _(j=j):
                    finish(j - _RING)
            @pl.when(gbase + j < cnt)
            def _(j=j):
                fetch(cb16[j], j % _RING)
        for j in range(_LANES - _RING, _LANES):
            @pl.when(gbase + j < cnt)
            def _(j=j):
                finish(j)
        return ()

    ngroups = (cnt + _LANES - 1) // _LANES
    lax.fori_loop(0, ngroups, select_group, (), unroll=False)


@jax.jit
def kernel(idx, emb_weight):
    return _sc_gather(emb_weight.T, idx.astype(jnp.int32))
